# Initial kernel scaffold; baseline (speedup 1.0000x reference)
import jax
import jax.numpy as jnp
from jax.experimental import pallas as pl


def kernel(agent_x, map_x, edge_index, params):
    raise NotImplementedError("write your pallas kernel here")


# speedup vs baseline: 57.7373x; 57.7373x over previous
import functools

import jax
import jax.numpy as jnp
from jax import lax
from jax.experimental import pallas as pl
from jax.experimental.pallas import tpu as pltpu
from jax.experimental.pallas import tpu_sc as plsc

N_AGENTS = 40000
N_MAP = 10000
N_NODES = N_AGENTS + N_MAP
N_PAD = 50048
N_EDGES = 800000
CHUNK = 128
N_CHUNKS_REAL = N_EDGES // CHUNK
W_CHUNKS = 196
N_CHUNKS_PAD = 32 * W_CHUNKS
GRP = 28
ROWS_PER_TILE = N_PAD // 16
AROWS_PER_TILE = N_AGENTS // 16
HID = 32
DEGW = 16


def _deg_body(e_hbm, zer, out, dstv, ones_v, deg_sh, sem):
    c = lax.axis_index("c")
    s = lax.axis_index("s")
    w = c * 16 + s

    r0 = s * ROWS_PER_TILE
    pltpu.sync_copy(zer.at[pl.ds(r0, ROWS_PER_TILE)],
                    deg_sh.at[pl.ds(r0, ROWS_PER_TILE)])

    one16 = jnp.ones((16,), jnp.float32)

    def fill(i, _):
        ones_v[i, pl.ds(0, 16)] = one16
        return 0

    lax.fori_loop(0, CHUNK, fill, 0, unroll=8)

    pltpu.sync_copy(e_hbm.at[1, pl.ds(w * W_CHUNKS, W_CHUNKS)], dstv)
    plsc.subcore_barrier()

    def body(j, _):
        pltpu.sync_copy(ones_v, deg_sh.at[dstv.at[j]], add=True)
        return 0

    lax.fori_loop(0, W_CHUNKS, body, 0)
    plsc.subcore_barrier()

    pltpu.sync_copy(deg_sh.at[pl.ds(r0, ROWS_PER_TILE)],
                    out.at[c, pl.ds(r0, ROWS_PER_TILE)])


def _deg_call(eidx, zer16):
    kern = functools.partial(
        pl.kernel,
        out_type=jax.ShapeDtypeStruct((2, N_PAD, DEGW), jnp.float32),
        mesh=plsc.VectorSubcoreMesh(core_axis_name="c", subcore_axis_name="s",
                                    num_cores=2, num_subcores=16),
        compiler_params=pltpu.CompilerParams(use_tc_tiling_on_sc=False),
        scratch_types=[
            pltpu.VMEM((W_CHUNKS, CHUNK), jnp.int32),
            pltpu.VMEM((CHUNK, DEGW), jnp.float32),
            pltpu.VMEM_SHARED((N_PAD, DEGW), jnp.float32),
            pltpu.SemaphoreType.DMA,
        ],
    )(_deg_body)
    return kern(eidx, zer16)


def _agg_body(xs_hbm, e_hbm, zer, out,
              srcv, dstv, rb0, rb1, yacc, sem0, sem1):
    c = lax.axis_index("c")
    s = lax.axis_index("s")
    w = c * 16 + s
    r0 = s * ROWS_PER_TILE

    pltpu.sync_copy(zer.at[pl.ds(r0, ROWS_PER_TILE)],
                    yacc.at[pl.ds(r0, ROWS_PER_TILE)])
    plsc.subcore_barrier()

    def group(gi, _):
        c0 = w * W_CHUNKS + gi * GRP
        pltpu.sync_copy(e_hbm.at[0, pl.ds(c0, GRP)], srcv)
        pltpu.sync_copy(e_hbm.at[1, pl.ds(c0, GRP)], dstv)
        pltpu.async_copy(xs_hbm.at[srcv.at[0]], rb0, sem0)

        def body(i, _):
            j0 = 2 * i
            pltpu.make_async_copy(xs_hbm.at[srcv.at[j0]], rb0, sem0).wait()
            pltpu.async_copy(xs_hbm.at[srcv.at[j0 + 1]], rb1, sem1)
            pltpu.sync_copy(rb0, yacc.at[dstv.at[j0]], add=True)
            pltpu.make_async_copy(xs_hbm.at[srcv.at[j0 + 1]], rb1,
                                  sem1).wait()

            @pl.when(j0 + 2 < GRP)
            def _():
                pltpu.async_copy(xs_hbm.at[srcv.at[j0 + 2]], rb0, sem0)

            pltpu.sync_copy(rb1, yacc.at[dstv.at[j0 + 1]], add=True)
            return 0

        lax.fori_loop(0, GRP // 2, body, 0)
        return 0

    lax.fori_loop(0, W_CHUNKS // GRP, group, 0)
    plsc.subcore_barrier()

    a0 = s * AROWS_PER_TILE
    pltpu.sync_copy(yacc.at[pl.ds(a0, AROWS_PER_TILE)],
                    out.at[c, pl.ds(a0, AROWS_PER_TILE)])


def _agg_call(xs, eidx, zer32):
    kern = functools.partial(
        pl.kernel,
        out_type=jax.ShapeDtypeStruct((2, N_AGENTS, HID), jnp.float32),
        mesh=plsc.VectorSubcoreMesh(core_axis_name="c", subcore_axis_name="s",
                                    num_cores=2, num_subcores=16),
        compiler_params=pltpu.CompilerParams(use_tc_tiling_on_sc=False),
        scratch_types=[
            pltpu.VMEM((GRP, CHUNK), jnp.int32),
            pltpu.VMEM((GRP, CHUNK), jnp.int32),
            pltpu.VMEM((CHUNK, HID), jnp.float32),
            pltpu.VMEM((CHUNK, HID), jnp.float32),
            pltpu.VMEM_SHARED((N_PAD, HID), jnp.float32),
            pltpu.SemaphoreType.DMA,
            pltpu.SemaphoreType.DMA,
        ],
    )(_agg_body)
    return kern(xs, eidx, zer32)


def _enc_body(xf_ref, wa_ref, wm_ref, ba_ref, bm_ref, deg_ref,
              xs_ref, dinv_ref):
    i = pl.program_id(0)
    rows = i * ROWS_PER_TILE + lax.broadcasted_iota(
        jnp.int32, (ROWS_PER_TILE, 1), 0)
    xf = xf_ref[...]
    ha = jnp.dot(xf, wa_ref[...], preferred_element_type=jnp.float32) \
        + ba_ref[...]
    hm = jnp.dot(xf, wm_ref[...], preferred_element_type=jnp.float32) \
        + bm_ref[...]
    x = jnp.where(rows < N_AGENTS, ha, hm)
    x = jnp.where(rows < N_NODES, x, 0.0)
    deg = deg_ref[0, :, :1] + deg_ref[1, :, :1] + 1.0
    dinv = lax.rsqrt(deg)
    xs_ref[...] = x * dinv
    dinv_ref[...] = dinv


def _enc_call(xf, wa, wm, ba, bm, deg_parts):
    grid = N_PAD // ROWS_PER_TILE
    return pl.pallas_call(
        _enc_body,
        grid=(grid,),
        in_specs=[
            pl.BlockSpec((ROWS_PER_TILE, 16), lambda i: (i, 0)),
            pl.BlockSpec((16, HID), lambda i: (0, 0)),
            pl.BlockSpec((16, HID), lambda i: (0, 0)),
            pl.BlockSpec((1, HID), lambda i: (0, 0)),
            pl.BlockSpec((1, HID), lambda i: (0, 0)),
            pl.BlockSpec((2, ROWS_PER_TILE, DEGW), lambda i: (0, i, 0)),
        ],
        out_specs=[
            pl.BlockSpec((ROWS_PER_TILE, HID), lambda i: (i, 0)),
            pl.BlockSpec((ROWS_PER_TILE, 1), lambda i: (i, 0)),
        ],
        out_shape=[
            jax.ShapeDtypeStruct((N_PAD, HID), jnp.float32),
            jax.ShapeDtypeStruct((N_PAD, 1), jnp.float32),
        ],
    )(xf, wa, wm, ba, bm, deg_parts)


def _fin_body(yp_ref, xs_ref, dinv_ref,
              wcz_ref, bcz_ref, wlz_ref, blz_ref,
              wch_ref, bch_ref, wlh_ref, blh_ref,
              wd1_ref, bd1_ref, wd2_ref, bd2_ref, out_ref):
    y = (yp_ref[0] + yp_ref[1] + xs_ref[...]) * dinv_ref[...]
    gz = jnp.dot(y, wcz_ref[...], preferred_element_type=jnp.float32) \
        + bcz_ref[...]
    z = jax.nn.sigmoid(
        jnp.dot(gz, wlz_ref[:HID, :], preferred_element_type=jnp.float32)
        + blz_ref[...])
    gh = jnp.dot(y, wch_ref[...], preferred_element_type=jnp.float32) \
        + bch_ref[...]
    ht = jnp.tanh(
        jnp.dot(gh, wlh_ref[:HID, :], preferred_element_type=jnp.float32)
        + blh_ref[...])
    h = jax.nn.relu((1.0 - z) * ht)
    d1 = jax.nn.relu(
        jnp.dot(h, wd1_ref[...], preferred_element_type=jnp.float32)
        + bd1_ref[...])
    out_ref[...] = jnp.dot(d1, wd2_ref[...],
                           preferred_element_type=jnp.float32) + bd2_ref[...]


def _fin_call(yp, xs, dinv, p):
    blk = 2000
    grid = N_AGENTS // blk
    full = lambda shp: pl.BlockSpec(shp, lambda i: tuple(0 for _ in shp))
    return pl.pallas_call(
        _fin_body,
        grid=(grid,),
        in_specs=[
            pl.BlockSpec((2, blk, HID), lambda i: (0, i, 0)),
            pl.BlockSpec((blk, HID), lambda i: (i, 0)),
            pl.BlockSpec((blk, 1), lambda i: (i, 0)),
            full((HID, HID)), full((1, HID)), full((2 * HID, HID)),
            full((1, HID)),
            full((HID, HID)), full((1, HID)), full((2 * HID, HID)),
            full((1, HID)),
            full((HID, 2 * HID)), full((1, 2 * HID)),
            full((2 * HID, 100)), full((1, 100)),
        ],
        out_specs=pl.BlockSpec((blk, 100), lambda i: (i, 0)),
        out_shape=jax.ShapeDtypeStruct((N_AGENTS, 100), jnp.float32),
    )(yp, xs, dinv,
      p['W_cz'], p['b_cz'].reshape(1, -1), p['W_lz'], p['b_lz'].reshape(1, -1),
      p['W_ch'], p['b_ch'].reshape(1, -1), p['W_lh'], p['b_lh'].reshape(1, -1),
      p['W_d1'], p['b_d1'].reshape(1, -1), p['W_d2'], p['b_d2'].reshape(1, -1))


@jax.jit
def kernel(agent_x, map_x, edge_index, params):
    p = params
    xf = jnp.zeros((N_PAD, 16), jnp.float32)
    xf = xf.at[:N_AGENTS, :9].set(agent_x)
    xf = xf.at[N_AGENTS:N_NODES, :6].set(map_x)
    wa = jnp.zeros((16, HID), jnp.float32).at[:9].set(p['W_ae'])
    wm = jnp.zeros((16, HID), jnp.float32).at[:6].set(p['W_me'])
    ba = p['b_ae'].reshape(1, -1)
    bm = p['b_me'].reshape(1, -1)

    eidx = jnp.pad(edge_index.reshape(2, N_CHUNKS_REAL, CHUNK),
                   ((0, 0), (0, N_CHUNKS_PAD - N_CHUNKS_REAL), (0, 0)),
                   constant_values=N_NODES)

    zer16 = jnp.zeros((N_PAD, DEGW), jnp.float32)
    zer32 = jnp.zeros((N_PAD, HID), jnp.float32)

    deg_parts = _deg_call(eidx, zer16)
    xs, dinv = _enc_call(xf, wa, wm, ba, bm, deg_parts)
    yp = _agg_call(xs, eidx, zer32)
    pred = _fin_call(yp, xs, dinv, p)
    return pred.reshape(-1, 50, 2)


# speedup vs baseline: 66.0201x; 1.1435x over previous
import functools

import jax
import jax.numpy as jnp
from jax import lax
from jax.experimental import pallas as pl
from jax.experimental.pallas import tpu as pltpu
from jax.experimental.pallas import tpu_sc as plsc

N_AGENTS = 40000
N_MAP = 10000
N_NODES = N_AGENTS + N_MAP
N_PAD = 50048
N_EDGES = 800000
CHUNK = 128
N_CHUNKS_REAL = N_EDGES // CHUNK
W_CHUNKS = 196
N_CHUNKS_PAD = 32 * W_CHUNKS
GRP = 28
ROWS_PER_TILE = N_PAD // 16
AROWS_PER_TILE = N_AGENTS // 16
HID = 32
DEGW = 16


def _deg_body(e_hbm, zer, out, dstv, ones_v, deg_sh, sems):
    c = lax.axis_index("c")
    s = lax.axis_index("s")
    w = c * 16 + s

    r0 = s * ROWS_PER_TILE
    pltpu.sync_copy(zer.at[pl.ds(r0, ROWS_PER_TILE)],
                    deg_sh.at[pl.ds(r0, ROWS_PER_TILE)])

    one16 = jnp.ones((16,), jnp.float32)

    def fill(i, _):
        ones_v[i, pl.ds(0, 16)] = one16
        return 0

    lax.fori_loop(0, CHUNK, fill, 0, unroll=8)

    pltpu.sync_copy(e_hbm.at[1, pl.ds(w * W_CHUNKS, W_CHUNKS)], dstv)
    plsc.subcore_barrier()

    base = w * W_CHUNKS

    def body(j, _):
        @pl.when(base + j < N_CHUNKS_REAL)
        def _():
            pltpu.sync_copy(ones_v, deg_sh.at[dstv.at[j]], add=True)
        return 0

    lax.fori_loop(0, W_CHUNKS, body, 0)
    plsc.subcore_barrier()

    pltpu.sync_copy(deg_sh.at[pl.ds(r0, ROWS_PER_TILE)],
                    out.at[c, pl.ds(r0, ROWS_PER_TILE)])


def _deg_call(eidx, zer16):
    kern = functools.partial(
        pl.kernel,
        out_type=jax.ShapeDtypeStruct((2, N_PAD, DEGW), jnp.float32),
        mesh=plsc.VectorSubcoreMesh(core_axis_name="c", subcore_axis_name="s",
                                    num_cores=2, num_subcores=16),
        compiler_params=pltpu.CompilerParams(use_tc_tiling_on_sc=False),
        scratch_types=[
            pltpu.VMEM((W_CHUNKS, CHUNK), jnp.int32),
            pltpu.VMEM((CHUNK, DEGW), jnp.float32),
            pltpu.VMEM_SHARED((N_PAD, DEGW), jnp.float32),
            [pltpu.SemaphoreType.DMA] * 4,
        ],
    )(_deg_body)
    return kern(eidx, zer16)


def _agg_body(xs_hbm, e_hbm, zer, out,
              srcv, dstv, rbs, yacc, gsem, ssem):
    c = lax.axis_index("c")
    s = lax.axis_index("s")
    w = c * 16 + s
    r0 = s * ROWS_PER_TILE

    pltpu.sync_copy(zer.at[pl.ds(r0, ROWS_PER_TILE)],
                    yacc.at[pl.ds(r0, ROWS_PER_TILE)])
    plsc.subcore_barrier()

    def group(gi, _):
        c0 = w * W_CHUNKS + gi * GRP
        pltpu.sync_copy(e_hbm.at[0, pl.ds(c0, GRP)], srcv)
        pltpu.sync_copy(e_hbm.at[1, pl.ds(c0, GRP)], dstv)

        def gath(j, b):
            return pltpu.make_async_copy(xs_hbm.at[srcv.at[j]], rbs[b],
                                         gsem[b])

        def scat(j, b):
            return pltpu.make_async_copy(rbs[b], yacc.at[dstv.at[j]],
                                         ssem[b])

        gath(0, 0).start()
        gath(1, 1).start()

        def body(i, _):
            for b in range(4):
                j = 4 * i + b
                gath(j, b).wait()

                @pl.when(c0 + j < N_CHUNKS_REAL)
                def _():
                    scat(j, b).start(add=True)

                @pl.when(j + 2 < GRP)
                def _():
                    @pl.when(jnp.logical_and(
                        j >= 2, c0 + j - 2 < N_CHUNKS_REAL))
                    def _():
                        scat(j - 2, (b + 2) % 4).wait()

                    gath(j + 2, (b + 2) % 4).start()
            return 0

        lax.fori_loop(0, GRP // 4, body, 0)
        for j in range(GRP - 4, GRP):
            @pl.when(c0 + j < N_CHUNKS_REAL)
            def _():
                scat(j, j % 4).wait()
        return 0

    lax.fori_loop(0, W_CHUNKS // GRP, group, 0)
    plsc.subcore_barrier()

    a0 = s * AROWS_PER_TILE
    pltpu.sync_copy(yacc.at[pl.ds(a0, AROWS_PER_TILE)],
                    out.at[c, pl.ds(a0, AROWS_PER_TILE)])


def _agg_call(xs, eidx, zer32):
    kern = functools.partial(
        pl.kernel,
        out_type=jax.ShapeDtypeStruct((2, N_AGENTS, HID), jnp.float32),
        mesh=plsc.VectorSubcoreMesh(core_axis_name="c", subcore_axis_name="s",
                                    num_cores=2, num_subcores=16),
        compiler_params=pltpu.CompilerParams(use_tc_tiling_on_sc=False),
        scratch_types=[
            pltpu.VMEM((GRP, CHUNK), jnp.int32),
            pltpu.VMEM((GRP, CHUNK), jnp.int32),
            [pltpu.VMEM((CHUNK, HID), jnp.float32)] * 4,
            pltpu.VMEM_SHARED((N_PAD, HID), jnp.float32),
            [pltpu.SemaphoreType.DMA] * 4,
            [pltpu.SemaphoreType.DMA] * 4,
        ],
    )(_agg_body)
    return kern(xs, eidx, zer32)


def _enc_body(xf_ref, wa_ref, wm_ref, ba_ref, bm_ref, deg_ref,
              xs_ref, dinv_ref):
    i = pl.program_id(0)
    rows = i * ROWS_PER_TILE + lax.broadcasted_iota(
        jnp.int32, (ROWS_PER_TILE, 1), 0)
    xf = xf_ref[...]
    ha = jnp.dot(xf, wa_ref[...], preferred_element_type=jnp.float32) \
        + ba_ref[...]
    hm = jnp.dot(xf, wm_ref[...], preferred_element_type=jnp.float32) \
        + bm_ref[...]
    x = jnp.where(rows < N_AGENTS, ha, hm)
    x = jnp.where(rows < N_NODES, x, 0.0)
    deg = deg_ref[0, :, :1] + deg_ref[1, :, :1] + 1.0
    dinv = lax.rsqrt(deg)
    xs_ref[...] = x * dinv
    dinv_ref[...] = dinv


def _enc_call(xf, wa, wm, ba, bm, deg_parts):
    grid = N_PAD // ROWS_PER_TILE
    return pl.pallas_call(
        _enc_body,
        grid=(grid,),
        in_specs=[
            pl.BlockSpec((ROWS_PER_TILE, 16), lambda i: (i, 0)),
            pl.BlockSpec((16, HID), lambda i: (0, 0)),
            pl.BlockSpec((16, HID), lambda i: (0, 0)),
            pl.BlockSpec((1, HID), lambda i: (0, 0)),
            pl.BlockSpec((1, HID), lambda i: (0, 0)),
            pl.BlockSpec((2, ROWS_PER_TILE, DEGW), lambda i: (0, i, 0)),
        ],
        out_specs=[
            pl.BlockSpec((ROWS_PER_TILE, HID), lambda i: (i, 0)),
            pl.BlockSpec((ROWS_PER_TILE, 1), lambda i: (i, 0)),
        ],
        out_shape=[
            jax.ShapeDtypeStruct((N_PAD, HID), jnp.float32),
            jax.ShapeDtypeStruct((N_PAD, 1), jnp.float32),
        ],
    )(xf, wa, wm, ba, bm, deg_parts)


def _fin_body(yp_ref, xs_ref, dinv_ref,
              wcz_ref, bcz_ref, wlz_ref, blz_ref,
              wch_ref, bch_ref, wlh_ref, blh_ref,
              wd1_ref, bd1_ref, wd2_ref, bd2_ref, out_ref):
    y = (yp_ref[0] + yp_ref[1] + xs_ref[...]) * dinv_ref[...]
    gz = jnp.dot(y, wcz_ref[...], preferred_element_type=jnp.float32) \
        + bcz_ref[...]
    z = jax.nn.sigmoid(
        jnp.dot(gz, wlz_ref[:HID, :], preferred_element_type=jnp.float32)
        + blz_ref[...])
    gh = jnp.dot(y, wch_ref[...], preferred_element_type=jnp.float32) \
        + bch_ref[...]
    ht = jnp.tanh(
        jnp.dot(gh, wlh_ref[:HID, :], preferred_element_type=jnp.float32)
        + blh_ref[...])
    h = jax.nn.relu((1.0 - z) * ht)
    d1 = jax.nn.relu(
        jnp.dot(h, wd1_ref[...], preferred_element_type=jnp.float32)
        + bd1_ref[...])
    out_ref[...] = jnp.dot(d1, wd2_ref[...],
                           preferred_element_type=jnp.float32) + bd2_ref[...]


def _fin_call(yp, xs, dinv, p):
    blk = 2000
    grid = N_AGENTS // blk
    full = lambda shp: pl.BlockSpec(shp, lambda i: tuple(0 for _ in shp))
    return pl.pallas_call(
        _fin_body,
        grid=(grid,),
        in_specs=[
            pl.BlockSpec((2, blk, HID), lambda i: (0, i, 0)),
            pl.BlockSpec((blk, HID), lambda i: (i, 0)),
            pl.BlockSpec((blk, 1), lambda i: (i, 0)),
            full((HID, HID)), full((1, HID)), full((2 * HID, HID)),
            full((1, HID)),
            full((HID, HID)), full((1, HID)), full((2 * HID, HID)),
            full((1, HID)),
            full((HID, 2 * HID)), full((1, 2 * HID)),
            full((2 * HID, 100)), full((1, 100)),
        ],
        out_specs=pl.BlockSpec((blk, 100), lambda i: (i, 0)),
        out_shape=jax.ShapeDtypeStruct((N_AGENTS, 100), jnp.float32),
    )(yp, xs, dinv,
      p['W_cz'], p['b_cz'].reshape(1, -1), p['W_lz'], p['b_lz'].reshape(1, -1),
      p['W_ch'], p['b_ch'].reshape(1, -1), p['W_lh'], p['b_lh'].reshape(1, -1),
      p['W_d1'], p['b_d1'].reshape(1, -1), p['W_d2'], p['b_d2'].reshape(1, -1))


@jax.jit
def kernel(agent_x, map_x, edge_index, params):
    p = params
    xf = jnp.zeros((N_PAD, 16), jnp.float32)
    xf = xf.at[:N_AGENTS, :9].set(agent_x)
    xf = xf.at[N_AGENTS:N_NODES, :6].set(map_x)
    wa = jnp.zeros((16, HID), jnp.float32).at[:9].set(p['W_ae'])
    wm = jnp.zeros((16, HID), jnp.float32).at[:6].set(p['W_me'])
    ba = p['b_ae'].reshape(1, -1)
    bm = p['b_me'].reshape(1, -1)

    eidx = jnp.pad(edge_index.reshape(2, N_CHUNKS_REAL, CHUNK),
                   ((0, 0), (0, N_CHUNKS_PAD - N_CHUNKS_REAL), (0, 0)),
                   constant_values=N_NODES)

    zer16 = jnp.zeros((N_PAD, DEGW), jnp.float32)
    zer32 = jnp.zeros((N_PAD, HID), jnp.float32)

    deg_parts = _deg_call(eidx, zer16)
    xs, dinv = _enc_call(xf, wa, wm, ba, bm, deg_parts)
    yp = _agg_call(xs, eidx, zer32)
    pred = _fin_call(yp, xs, dinv, p)
    return pred.reshape(-1, 50, 2)
